# baseline (speedup 1.0000x reference)
import functools

import jax
import jax.numpy as jnp
from jax import lax
from jax.experimental import pallas as pl
from jax.experimental.pallas import tpu as pltpu
from jax.experimental.pallas import tpu_sc as plsc

_B = 16384
_F = 64
_FP = _F // 2
_NC = 2
_NS = 16
_L = 16
_NW = _NC * _NS
_BPW = _B // _NW
_CHUNK = 128
_NCHUNK = _BPW // _CHUNK
_TBLK = 65536
_QB = _TBLK // 4
_TSH = _TBLK.bit_length() - 1
_QSH = _QB.bit_length() - 1
_BBLK = 16384


def _mf_body(ue_t, me_t, ub_t, mb_t, uidx_hbm, midx_hbm, out_hbm,
             uidx_v, midx_v, ue_v0, ue_v1, me_v0, me_v1, ub_s, mb_s,
             urow_v, mrow_v, ubrow_v, mbrow_v, out_v,
             sem0, sem1, sem2, sem3, semb):
    sems = [sem0, sem1, sem2, sem3]
    uebufs = [ue_v0, ue_v1]
    mebufs = [me_v0, me_v1]
    wid = lax.axis_index("s") * _NC + lax.axis_index("c")

    pltpu.sync_copy(uidx_hbm.at[wid], uidx_v)
    pltpu.sync_copy(midx_hbm.at[wid], midx_v)

    for c in range(_NCHUNK):
        for k in range(_CHUNK // _L):
            sl = pl.ds(k * _L, _L)
            uv = uidx_v[c, sl]
            mv = midx_v[c, sl]
            urow_v[c, sl] = lax.bitwise_or(
                lax.shift_left(lax.shift_right_logical(uv, _TSH), _QSH),
                lax.bitwise_and(uv, _QB - 1))
            mrow_v[c, sl] = lax.bitwise_or(
                lax.shift_left(lax.shift_right_logical(mv, _TSH), _QSH),
                lax.bitwise_and(mv, _QB - 1))
            ubrow_v[c, sl] = lax.shift_right_logical(uv, 4)
            mbrow_v[c, sl] = lax.shift_right_logical(mv, 4)

    emb_cp = [None] * _NCHUNK

    def fire(c):
        emb_cp[c] = (
            pltpu.async_copy(ue_t.at[urow_v.at[c]], uebufs[c % 2], sems[c]),
            pltpu.async_copy(me_t.at[mrow_v.at[c]], mebufs[c % 2], sems[c]),
        )

    fire(0)
    fire(1)
    bias_cp = []
    for c in range(_NCHUNK):
        dst = pl.ds(c * _CHUNK, _CHUNK)
        bias_cp.append(pltpu.async_copy(ub_t.at[ubrow_v.at[c]], ub_s.at[dst], semb))
        bias_cp.append(pltpu.async_copy(mb_t.at[mbrow_v.at[c]], mb_s.at[dst], semb))
    for cp in bias_cp:
        cp.wait()

    for c in range(_NCHUNK):
        ucp, mcp = emb_cp[c]
        ucp.wait()
        mcp.wait()
        ue_v = uebufs[c % 2]
        me_v = mebufs[c % 2]

        def group_body(g, carry, c=c, ue_v=ue_v, me_v=me_v):
            lrow = g * _L + lax.iota(jnp.int32, _L)
            row = c * _CHUNK + lrow
            pos = g * _L + lax.iota(jnp.int32, _L)
            cvec = jnp.full((_L,), c, jnp.int32)
            uvals = plsc.load_gather(uidx_v, [cvec, pos])
            mvals = plsc.load_gather(midx_v, [cvec, pos])
            uoff = lax.shift_left(
                lax.bitwise_and(lax.shift_right_logical(uvals, _QSH), 3), 5)
            moff = lax.shift_left(
                lax.bitwise_and(lax.shift_right_logical(mvals, _QSH), 3), 5)
            acc = jnp.zeros((_L,), jnp.float32)
            for j in range(_FP):
                vu = plsc.load_gather(ue_v, [lrow, uoff + j])
                vm = plsc.load_gather(me_v, [lrow, moff + j])
                p = plsc.bitcast(vu, jnp.bfloat16) * plsc.bitcast(vm, jnp.bfloat16)
                pa, pb = plsc.unpack(p, format=plsc.PackFormat.INTERLEAVED)
                acc = acc + pa + pb
            ubv = plsc.load_gather(ub_s, [row, lax.bitwise_and(uvals, _L - 1)])
            mbv = plsc.load_gather(mb_s, [row, lax.bitwise_and(mvals, _L - 1)])
            out_v[pl.ds(c * _CHUNK + g * _L, _L)] = acc + ubv + mbv + 3.5
            return carry

        lax.fori_loop(0, _CHUNK // _L, group_body, 0)
        if c + 2 < _NCHUNK:
            fire(c + 2)

    pltpu.sync_copy(out_v, out_hbm.at[pl.ds(wid * _BPW, _BPW)])


_mf_kernel = functools.partial(
    pl.kernel,
    out_type=jax.ShapeDtypeStruct((_B,), jnp.float32),
    mesh=plsc.VectorSubcoreMesh(core_axis_name="c", subcore_axis_name="s"),
    compiler_params=pltpu.CompilerParams(
        needs_layout_passes=False, use_tc_tiling_on_sc=False),
    scratch_types=[
        pltpu.VMEM((_NCHUNK, _CHUNK), jnp.int32),
        pltpu.VMEM((_NCHUNK, _CHUNK), jnp.int32),
        pltpu.VMEM((_CHUNK, 128), jnp.int32),
        pltpu.VMEM((_CHUNK, 128), jnp.int32),
        pltpu.VMEM((_CHUNK, 128), jnp.int32),
        pltpu.VMEM((_CHUNK, 128), jnp.int32),
        pltpu.VMEM((_BPW, _L), jnp.float32),
        pltpu.VMEM((_BPW, _L), jnp.float32),
        pltpu.VMEM((_NCHUNK, _CHUNK), jnp.int32),
        pltpu.VMEM((_NCHUNK, _CHUNK), jnp.int32),
        pltpu.VMEM((_NCHUNK, _CHUNK), jnp.int32),
        pltpu.VMEM((_NCHUNK, _CHUNK), jnp.int32),
        pltpu.VMEM((_BPW,), jnp.float32),
        pltpu.SemaphoreType.DMA,
        pltpu.SemaphoreType.DMA,
        pltpu.SemaphoreType.DMA,
        pltpu.SemaphoreType.DMA,
        pltpu.SemaphoreType.DMA,
    ],
)(_mf_body)


def _tp_body(in_ref, out_ref):
    x = in_ref[...]
    lo = jax.lax.bitcast_convert_type(
        x[:_FP, :].astype(jnp.bfloat16), jnp.uint16).astype(jnp.uint32)
    hi = jax.lax.bitcast_convert_type(
        x[_FP:, :].astype(jnp.bfloat16), jnp.uint16).astype(jnp.uint32)
    p = jax.lax.bitcast_convert_type(lo | (hi << jnp.uint32(16)), jnp.int32)
    m = jnp.concatenate([p[:, k * _QB:(k + 1) * _QB] for k in range(4)], axis=0)
    out_ref[...] = m.T


def _bias_body(in_ref, out_ref):
    x = in_ref[...]
    out_ref[...] = jnp.concatenate(
        [x[:, k * 128:(k + 1) * 128] for k in range(128)], axis=0)


def _bias_flat_tc(bias_t):
    n = bias_t.shape[1]
    nblk = (n + _BBLK - 1) // _BBLK
    out = pl.pallas_call(
        _bias_body,
        grid=(nblk,),
        in_specs=[pl.BlockSpec((1, _BBLK), lambda i: (0, i))],
        out_specs=pl.BlockSpec((128, 128), lambda i: (i, 0)),
        out_shape=jax.ShapeDtypeStruct((nblk * 128, 128), jnp.float32),
    )(bias_t)
    return out.reshape(nblk * _BBLK // _L, _L)


def _relayout_pack_tc(table_t):
    n = table_t.shape[1]
    nblk = (n + _TBLK - 1) // _TBLK
    return pl.pallas_call(
        _tp_body,
        grid=(nblk,),
        in_specs=[pl.BlockSpec((_F, _TBLK), lambda i: (0, i))],
        out_specs=pl.BlockSpec((_QB, 128), lambda i: (i, 0)),
        out_shape=jax.ShapeDtypeStruct((nblk * _QB, 128), jnp.int32),
    )(table_t)


@jax.jit
def kernel(user_emb, movie_emb, user_bias, movie_bias, user_indices, movie_indices):
    uidx = user_indices.astype(jnp.int32).reshape(_NW, _NCHUNK, _CHUNK)
    midx = movie_indices.astype(jnp.int32).reshape(_NW, _NCHUNK, _CHUNK)
    ub = _bias_flat_tc(user_bias.T)
    mb = _bias_flat_tc(movie_bias.T)
    ue_p = _relayout_pack_tc(user_emb.T)
    me_p = _relayout_pack_tc(movie_emb.T)
    return _mf_kernel(ue_p, me_p, ub, mb, uidx, midx)


# speedup vs baseline: 1.1905x; 1.1905x over previous
import functools

import jax
import jax.numpy as jnp
from jax import lax
from jax.experimental import pallas as pl
from jax.experimental.pallas import tpu as pltpu
from jax.experimental.pallas import tpu_sc as plsc

_B = 16384
_F = 64
_FP = _F // 2
_NC = 2
_NS = 16
_L = 16
_NW = _NC * _NS
_BPW = _B // _NW
_CHUNK = 128
_NCHUNK = _BPW // _CHUNK
_TBLK = 65536
_QB = _TBLK // 4
_TSH = _TBLK.bit_length() - 1
_QSH = _QB.bit_length() - 1
_BBLK = 16384


def _mf_body(ue_t, me_t, ub_t, mb_t, uidx_hbm, midx_hbm, out_hbm,
             uidx_v, midx_v, ue_v0, ue_v1, me_v0, me_v1, ub_s, mb_s,
             urow_v, mrow_v, ubrow_v, mbrow_v, out_v,
             sem0, sem1, sem2, sem3, semb):
    sems = [sem0, sem1, sem2, sem3]
    uebufs = [ue_v0, ue_v1]
    mebufs = [me_v0, me_v1]
    wid = lax.axis_index("s") * _NC + lax.axis_index("c")

    pltpu.sync_copy(uidx_hbm.at[wid], uidx_v)
    pltpu.sync_copy(midx_hbm.at[wid], midx_v)

    for c in range(_NCHUNK):
        for k in range(_CHUNK // _L):
            sl = pl.ds(k * _L, _L)
            uv = uidx_v[c, sl]
            mv = midx_v[c, sl]
            urow_v[c, sl] = lax.bitwise_or(
                lax.shift_left(lax.shift_right_logical(uv, _TSH), _QSH),
                lax.bitwise_and(uv, _QB - 1))
            mrow_v[c, sl] = lax.bitwise_or(
                lax.shift_left(lax.shift_right_logical(mv, _TSH), _QSH),
                lax.bitwise_and(mv, _QB - 1))
            ubrow_v[c, sl] = lax.shift_right_logical(uv, 4)
            mbrow_v[c, sl] = lax.shift_right_logical(mv, 4)

    emb_cp = [None] * _NCHUNK

    def fire(c):
        emb_cp[c] = (
            pltpu.async_copy(ue_t.at[urow_v.at[c]], uebufs[c % 2], sems[c]),
            pltpu.async_copy(me_t.at[mrow_v.at[c]], mebufs[c % 2], sems[c]),
        )

    fire(0)
    fire(1)
    bias_cp = []
    for c in range(_NCHUNK):
        dst = pl.ds(c * _CHUNK, _CHUNK)
        bias_cp.append(pltpu.async_copy(ub_t.at[ubrow_v.at[c]], ub_s.at[dst], semb))
        bias_cp.append(pltpu.async_copy(mb_t.at[mbrow_v.at[c]], mb_s.at[dst], semb))
    for cp in bias_cp:
        cp.wait()

    for c in range(_NCHUNK):
        ucp, mcp = emb_cp[c]
        ucp.wait()
        mcp.wait()
        ue_v = uebufs[c % 2]
        me_v = mebufs[c % 2]

        def group_body(g, carry, c=c, ue_v=ue_v, me_v=me_v):
            lrow = g * _L + lax.iota(jnp.int32, _L)
            row = c * _CHUNK + lrow
            pos = g * _L + lax.iota(jnp.int32, _L)
            cvec = jnp.full((_L,), c, jnp.int32)
            uvals = plsc.load_gather(uidx_v, [cvec, pos])
            mvals = plsc.load_gather(midx_v, [cvec, pos])
            uoff = lax.shift_left(
                lax.bitwise_and(lax.shift_right_logical(uvals, _QSH), 3), 5)
            moff = lax.shift_left(
                lax.bitwise_and(lax.shift_right_logical(mvals, _QSH), 3), 5)
            acc = jnp.zeros((_L,), jnp.float32)
            for j in range(_FP):
                vu = plsc.load_gather(ue_v, [lrow, uoff + j])
                vm = plsc.load_gather(me_v, [lrow, moff + j])
                p = plsc.bitcast(vu, jnp.bfloat16) * plsc.bitcast(vm, jnp.bfloat16)
                pa, pb = plsc.unpack(p, format=plsc.PackFormat.INTERLEAVED)
                acc = acc + pa + pb
            ubv = plsc.load_gather(ub_s, [row, lax.bitwise_and(uvals, _L - 1)])
            mbv = plsc.load_gather(mb_s, [row, lax.bitwise_and(mvals, _L - 1)])
            out_v[pl.ds(c * _CHUNK + g * _L, _L)] = acc + ubv + mbv + 3.5
            return carry

        lax.fori_loop(0, _CHUNK // _L, group_body, 0)
        if c + 2 < _NCHUNK:
            fire(c + 2)

    pltpu.sync_copy(out_v, out_hbm.at[pl.ds(wid * _BPW, _BPW)])


_mf_kernel = functools.partial(
    pl.kernel,
    out_type=jax.ShapeDtypeStruct((_B,), jnp.float32),
    mesh=plsc.VectorSubcoreMesh(core_axis_name="c", subcore_axis_name="s"),
    compiler_params=pltpu.CompilerParams(
        needs_layout_passes=False, use_tc_tiling_on_sc=False),
    scratch_types=[
        pltpu.VMEM((_NCHUNK, _CHUNK), jnp.int32),
        pltpu.VMEM((_NCHUNK, _CHUNK), jnp.int32),
        pltpu.VMEM((_CHUNK, 128), jnp.int32),
        pltpu.VMEM((_CHUNK, 128), jnp.int32),
        pltpu.VMEM((_CHUNK, 128), jnp.int32),
        pltpu.VMEM((_CHUNK, 128), jnp.int32),
        pltpu.VMEM((_BPW, _L), jnp.float32),
        pltpu.VMEM((_BPW, _L), jnp.float32),
        pltpu.VMEM((_NCHUNK, _CHUNK), jnp.int32),
        pltpu.VMEM((_NCHUNK, _CHUNK), jnp.int32),
        pltpu.VMEM((_NCHUNK, _CHUNK), jnp.int32),
        pltpu.VMEM((_NCHUNK, _CHUNK), jnp.int32),
        pltpu.VMEM((_BPW,), jnp.float32),
        pltpu.SemaphoreType.DMA,
        pltpu.SemaphoreType.DMA,
        pltpu.SemaphoreType.DMA,
        pltpu.SemaphoreType.DMA,
        pltpu.SemaphoreType.DMA,
    ],
)(_mf_body)


def _tp_body(in_ref, bias_ref, out_ref, bias_out_ref):
    x = in_ref[...]
    lo = jax.lax.bitcast_convert_type(
        x[:_FP, :].astype(jnp.bfloat16), jnp.uint16).astype(jnp.uint32)
    hi = jax.lax.bitcast_convert_type(
        x[_FP:, :].astype(jnp.bfloat16), jnp.uint16).astype(jnp.uint32)
    p = jax.lax.bitcast_convert_type(lo | (hi << jnp.uint32(16)), jnp.int32)
    m = jnp.concatenate([p[:, k * _QB:(k + 1) * _QB] for k in range(4)], axis=0)
    out_ref[...] = m.T
    b = bias_ref[...]
    bias_out_ref[...] = jnp.concatenate(
        [b[:, k * 128:(k + 1) * 128] for k in range(_TBLK // 128)], axis=0)


def _relayout_pack_tc(table_t, bias_t):
    n = table_t.shape[1]
    nblk = (n + _TBLK - 1) // _TBLK
    packed, bias_flat = pl.pallas_call(
        _tp_body,
        grid=(nblk,),
        in_specs=[
            pl.BlockSpec((_F, _TBLK), lambda i: (0, i)),
            pl.BlockSpec((1, _TBLK), lambda i: (0, i)),
        ],
        out_specs=(
            pl.BlockSpec((_QB, 128), lambda i: (i, 0)),
            pl.BlockSpec((_TBLK // 128, 128), lambda i: (i, 0)),
        ),
        out_shape=(
            jax.ShapeDtypeStruct((nblk * _QB, 128), jnp.int32),
            jax.ShapeDtypeStruct((nblk * _TBLK // 128, 128), jnp.float32),
        ),
    )(table_t, bias_t)
    return packed, bias_flat.reshape(nblk * _TBLK // _L, _L)


@jax.jit
def kernel(user_emb, movie_emb, user_bias, movie_bias, user_indices, movie_indices):
    uidx = user_indices.astype(jnp.int32).reshape(_NW, _NCHUNK, _CHUNK)
    midx = movie_indices.astype(jnp.int32).reshape(_NW, _NCHUNK, _CHUNK)
    ue_p, ub = _relayout_pack_tc(user_emb.T, user_bias.T)
    me_p, mb = _relayout_pack_tc(movie_emb.T, movie_bias.T)
    return _mf_kernel(ue_p, me_p, ub, mb, uidx, midx)
